# baseline (speedup 1.0000x reference)
import functools
import jax
import jax.numpy as jnp
from jax import lax
from jax.experimental import pallas as pl
from jax.experimental.pallas import tpu as pltpu
from jax.experimental.pallas import tpu_sc as plsc

_SRC_PAD = 0
_TGT_PAD = 0
_EMBED = 32
_NW = 32
_BT = 128
_L = 16


def _mask_rows(idx_row, rows_v, pad):
    zeros = jnp.zeros((_L,), jnp.float32)
    iota = lax.iota(jnp.int32, _L)
    for g in range(_BT // _L):
        idxc = idx_row[pl.ds(g * _L, _L)]
        padmask = idxc == pad
        rowids = g * _L + iota
        for col in range(_EMBED):
            colids = jnp.full((_L,), col, jnp.int32)
            plsc.store_scatter(rows_v, [rowids, colids], zeros, mask=padmask)


def _lookup(idx_hbm, table_hbm, out_hbm, idx_raw, idx_v, rows0, rows1,
            gsem0, gsem1, wsem0, wsem1, wid, nseq, pad):
    pltpu.sync_copy(idx_hbm.at[pl.ds(wid * _BT, _BT)], idx_raw)

    def idx_t_body(s, carry):
        cid = jnp.full((_L,), s, jnp.int32)
        for g in range(_BT // _L):
            rid = g * _L + lax.iota(jnp.int32, _L)
            idx_v[s, pl.ds(g * _L, _L)] = plsc.load_gather(idx_raw, [rid, cid])
        return carry

    lax.fori_loop(0, nseq, idx_t_body, 0)
    npairs = nseq // 2

    def gather(s, rows, gsem):
        pltpu.async_copy(table_hbm.at[idx_v.at[s]], rows, gsem)

    def wait_gather(s, rows, gsem):
        pltpu.make_async_copy(table_hbm.at[idx_v.at[s]], rows, gsem).wait()

    def write(s, rows, wsem):
        pltpu.async_copy(rows, out_hbm.at[pl.ds(wid * _BT, _BT), s], wsem)

    def wait_write(s, rows, wsem):
        pltpu.make_async_copy(
            rows, out_hbm.at[pl.ds(wid * _BT, _BT), s], wsem).wait()

    gather(0, rows0, gsem0)

    def pair(jj, carry):
        s0 = 2 * jj
        s1 = 2 * jj + 1
        wait_gather(s0, rows0, gsem0)
        _mask_rows(idx_v.at[s0], rows0, pad)

        @pl.when(jj >= 1)
        def _():
            wait_write(s1 - 2, rows1, wsem1)

        gather(s1, rows1, gsem1)
        write(s0, rows0, wsem0)

        wait_gather(s1, rows1, gsem1)
        _mask_rows(idx_v.at[s1], rows1, pad)

        @pl.when(jj <= npairs - 2)
        def _():
            wait_write(s0, rows0, wsem0)
            gather(s0 + 2, rows0, gsem0)

        write(s1, rows1, wsem1)
        return carry

    lax.fori_loop(0, npairs, pair, 0)
    wait_write(nseq - 2, rows0, wsem0)
    wait_write(nseq - 1, rows1, wsem1)


def _make_kernel(nseq, nbatch, pad):
    @functools.partial(
        pl.kernel,
        out_type=jax.ShapeDtypeStruct((nbatch, nseq, _EMBED), jnp.float32),
        mesh=plsc.VectorSubcoreMesh(core_axis_name="c", subcore_axis_name="s"),
        compiler_params=pltpu.CompilerParams(
            needs_layout_passes=False, use_tc_tiling_on_sc=False),
        scratch_types=[
            pltpu.VMEM((_BT, nseq), jnp.int32),
            pltpu.VMEM((nseq, _BT), jnp.int32),
            pltpu.VMEM((_BT, _EMBED), jnp.float32),
            pltpu.VMEM((_BT, _EMBED), jnp.float32),
            pltpu.SemaphoreType.DMA,
            pltpu.SemaphoreType.DMA,
            pltpu.SemaphoreType.DMA,
            pltpu.SemaphoreType.DMA,
        ],
    )
    def emb_kernel(idx, table, out, idx_raw, idx_v, rows0, rows1,
                   gsem0, gsem1, wsem0, wsem1):
        wid = lax.axis_index("s") * 2 + lax.axis_index("c")
        _lookup(idx, table, out, idx_raw, idx_v, rows0, rows1,
                gsem0, gsem1, wsem0, wsem1, wid, nseq, pad)

    return emb_kernel


def _tc_row_major(weight_t):
    emb, v = weight_t.shape
    lanes_in = 512
    rows_out = lanes_in // 4
    grid = (v + lanes_in - 1) // lanes_in

    def body(x_ref, o_ref):
        y = x_ref[...].T
        r_iota = lax.broadcasted_iota(jnp.int32, (rows_out, lanes_in), 0)
        k_iota = lax.broadcasted_iota(jnp.int32, (rows_out, lanes_in), 1)
        for j in range(4):
            sel = (k_iota == 4 * r_iota + j).astype(jnp.float32)
            z = jax.lax.dot_general(
                sel, y, (((1,), (0,)), ((), ())),
                preferred_element_type=jnp.float32)
            o_ref[:, j * emb:(j + 1) * emb] = z

    return pl.pallas_call(
        body,
        grid=(grid,),
        in_specs=[pl.BlockSpec((emb, lanes_in), lambda i: (0, i))],
        out_specs=pl.BlockSpec((rows_out, 128), lambda i: (i, 0)),
        out_shape=jax.ShapeDtypeStruct((v // 4, 128), jnp.float32),
    )(weight_t)


def kernel(src_indices, tgt_indices, source_weight, target_weight):
    batch, seq = src_indices.shape
    vocab_s, emb = source_weight.shape
    vocab_t, _ = target_weight.shape
    src_w = _tc_row_major(jnp.transpose(source_weight)).reshape(vocab_s, emb)
    tgt_w = _tc_row_major(jnp.transpose(target_weight)).reshape(vocab_t, emb)
    src_out = _make_kernel(seq, batch, _SRC_PAD)(src_indices, src_w)
    tgt_out = _make_kernel(seq, batch, _TGT_PAD)(tgt_indices, tgt_w)
    return (src_out, tgt_out)


# speedup vs baseline: 2.4708x; 2.4708x over previous
import functools
import jax
import jax.numpy as jnp
from jax import lax
from jax.experimental import pallas as pl
from jax.experimental.pallas import tpu as pltpu
from jax.experimental.pallas import tpu_sc as plsc

_SRC_PAD = 0
_TGT_PAD = 0
_EMBED = 32
_NW = 32
_BT = 128
_L = 16


def _mask_rows(idx_row, rows_v, pad):
    zeros = jnp.zeros((_L,), jnp.float32)
    iota = lax.iota(jnp.int32, _L)
    for g in range(_BT // _L):
        idxc = idx_row[pl.ds(g * _L, _L)]
        padmask = idxc == pad
        rowids = g * _L + iota
        for col in range(_EMBED):
            colids = jnp.full((_L,), col, jnp.int32)
            plsc.store_scatter(rows_v, [rowids, colids], zeros, mask=padmask)


def _lookup(idx_hbm, table_hbm, out_hbm, idx_raw, idx_v, rows0, rows1,
            gsem0, gsem1, wsem0, wsem1, wid, nseq, pad):
    pltpu.sync_copy(idx_hbm.at[pl.ds(wid * _BT, _BT)], idx_raw)

    def idx_t_body(s, carry):
        cid = jnp.full((_L,), s, jnp.int32)
        for g in range(_BT // _L):
            rid = g * _L + lax.iota(jnp.int32, _L)
            idx_v[s, pl.ds(g * _L, _L)] = plsc.load_gather(idx_raw, [rid, cid])
        return carry

    lax.fori_loop(0, nseq, idx_t_body, 0)
    npairs = nseq // 2

    def gather(s, rows, gsem):
        pltpu.async_copy(table_hbm.at[idx_v.at[s]], rows, gsem)

    def wait_gather(s, rows, gsem):
        pltpu.make_async_copy(table_hbm.at[idx_v.at[s]], rows, gsem).wait()

    def write(s, rows, wsem):
        pltpu.async_copy(rows, out_hbm.at[pl.ds(wid * _BT, _BT), s], wsem)

    def wait_write(s, rows, wsem):
        pltpu.make_async_copy(
            rows, out_hbm.at[pl.ds(wid * _BT, _BT), s], wsem).wait()

    gather(0, rows0, gsem0)

    def pair(jj, carry):
        s0 = 2 * jj
        s1 = 2 * jj + 1
        wait_gather(s0, rows0, gsem0)
        _mask_rows(idx_v.at[s0], rows0, pad)

        @pl.when(jj >= 1)
        def _():
            wait_write(s1 - 2, rows1, wsem1)

        gather(s1, rows1, gsem1)
        write(s0, rows0, wsem0)

        wait_gather(s1, rows1, gsem1)
        _mask_rows(idx_v.at[s1], rows1, pad)

        @pl.when(jj <= npairs - 2)
        def _():
            wait_write(s0, rows0, wsem0)
            gather(s0 + 2, rows0, gsem0)

        write(s1, rows1, wsem1)
        return carry

    lax.fori_loop(0, npairs, pair, 0)
    wait_write(nseq - 2, rows0, wsem0)
    wait_write(nseq - 1, rows1, wsem1)


def _make_kernel(nseq, nbatch, pad):
    @functools.partial(
        pl.kernel,
        out_type=jax.ShapeDtypeStruct((nbatch, nseq, _EMBED), jnp.float32),
        mesh=plsc.VectorSubcoreMesh(core_axis_name="c", subcore_axis_name="s"),
        compiler_params=pltpu.CompilerParams(
            needs_layout_passes=False, use_tc_tiling_on_sc=False),
        scratch_types=[
            pltpu.VMEM((_BT, nseq), jnp.int32),
            pltpu.VMEM((nseq, _BT), jnp.int32),
            pltpu.VMEM((_BT, _EMBED), jnp.float32),
            pltpu.VMEM((_BT, _EMBED), jnp.float32),
            pltpu.SemaphoreType.DMA,
            pltpu.SemaphoreType.DMA,
            pltpu.SemaphoreType.DMA,
            pltpu.SemaphoreType.DMA,
        ],
    )
    def emb_kernel(idx, table, out, idx_raw, idx_v, rows0, rows1,
                   gsem0, gsem1, wsem0, wsem1):
        wid = lax.axis_index("s") * 2 + lax.axis_index("c")
        _lookup(idx, table, out, idx_raw, idx_v, rows0, rows1,
                gsem0, gsem1, wsem0, wsem1, wid, nseq, pad)

    return emb_kernel


def kernel(src_indices, tgt_indices, source_weight, target_weight):
    batch, seq = src_indices.shape
    src_out = _make_kernel(seq, batch, _SRC_PAD)(src_indices, source_weight)
    tgt_out = _make_kernel(seq, batch, _TGT_PAD)(tgt_indices, target_weight)
    return (src_out, tgt_out)
